# Initial kernel scaffold; baseline (speedup 1.0000x reference)
import jax
import jax.numpy as jnp
from jax.experimental import pallas as pl


def kernel(edge_index, x, r, edge_type, edge_type_nhop, W0, a0, W1, a1, W2, a2, W3, a3, W_out, a_out, W_rel):
    raise NotImplementedError("write your pallas kernel here")


# speedup vs baseline: 5.6319x; 5.6319x over previous
import functools

import jax
import jax.numpy as jnp
from jax import lax
from jax.experimental import pallas as pl
from jax.experimental.pallas import tpu as pltpu
from jax.experimental.pallas import tpu_sc as plsc

N = 10000
E1 = 256000
EN = 64000
ET = E1 + EN
NREL = 200
TAB = 208
NW = 32
F32 = jnp.float32
I32 = jnp.int32

_SC_PARAMS = dict(
    mesh=plsc.VectorSubcoreMesh(core_axis_name="c", subcore_axis_name="s"),
    compiler_params=pltpu.CompilerParams(needs_layout_passes=False,
                                         use_tc_tiling_on_sc=False),
)


def _prep_body(W0, a0, W1, a1, W2, a2, W3, a3, W_out, a_out, W_rel, r,
               Md1, Ms1, Ma1, tab1, ra1, Ma2, tab2, ra2, r2):
    Ws = [W0[...], W1[...], W2[...], W3[...]]
    As = [a0[...], a1[...], a2[...], a3[...]]
    Md1[...] = jnp.concatenate([w[128:256] for w in Ws], axis=1)
    Ms1[...] = jnp.concatenate([w[0:128] for w in Ws], axis=1)
    Ma1[...] = jnp.concatenate(
        [jnp.dot(w[0:128], a, preferred_element_type=F32) for w, a in zip(Ws, As)]
        + [jnp.dot(w[128:256], a, preferred_element_type=F32) for w, a in zip(Ws, As)],
        axis=1)
    rr = r[...]
    wec = jnp.concatenate([w[256:272] for w in Ws], axis=1)
    t1v = jnp.dot(rr, wec, preferred_element_type=F32)
    z8 = jnp.zeros((8, 128), F32)
    tab1[...] = jnp.concatenate([t1v, z8], axis=0)
    ra1v = jnp.concatenate(
        [jnp.dot(t1v[:, 32 * k:32 * k + 32], As[k], preferred_element_type=F32)
         for k in range(4)], axis=1)
    ra1[...] = jnp.concatenate([ra1v, jnp.zeros((8, 4), F32)], axis=0)
    wo = W_out[...]
    ao = a_out[...]
    Ma2[...] = jnp.concatenate(
        [jnp.dot(wo[0:128], ao, preferred_element_type=F32),
         jnp.dot(wo[128:256], ao, preferred_element_type=F32),
         jnp.zeros((128, 6), F32)], axis=1)
    r2v = jnp.dot(rr, W_rel[...], preferred_element_type=F32)
    r2[...] = r2v
    t2v = jnp.dot(r2v, wo[256:384], preferred_element_type=F32)
    tab2[...] = jnp.concatenate([t2v, z8], axis=0)
    ra2v = jnp.dot(t2v, ao, preferred_element_type=F32)
    ra2[...] = jnp.concatenate(
        [jnp.concatenate([ra2v, jnp.zeros((200, 3), F32)], axis=1),
         jnp.zeros((8, 4), F32)], axis=0)


def _prep(W0, a0, W1, a1, W2, a2, W3, a3, W_out, a_out, W_rel, r):
    outs = (
        jax.ShapeDtypeStruct((128, 128), F32),
        jax.ShapeDtypeStruct((128, 128), F32),
        jax.ShapeDtypeStruct((128, 8), F32),
        jax.ShapeDtypeStruct((TAB, 128), F32),
        jax.ShapeDtypeStruct((TAB, 4), F32),
        jax.ShapeDtypeStruct((128, 8), F32),
        jax.ShapeDtypeStruct((TAB, 128), F32),
        jax.ShapeDtypeStruct((TAB, 4), F32),
        jax.ShapeDtypeStruct((200, 128), F32),
    )
    return pl.pallas_call(_prep_body, out_shape=outs)(
        W0, a0, W1, a1, W2, a2, W3, a3, W_out, a_out, W_rel, r)


def _proj_body(x, Md, Ms, Ma, xd, xs, scalT):
    xv = x[...]
    xd[...] = jnp.dot(xv, Md[...], preferred_element_type=F32)
    xs[...] = jnp.dot(xv, Ms[...], preferred_element_type=F32)
    scalT[...] = lax.dot_general(Ma[...], xv, (((0,), (1,)), ((), ())),
                                 preferred_element_type=F32)


def _proj(x, Md, Ms, Ma):
    outs = (
        jax.ShapeDtypeStruct((N, 128), F32),
        jax.ShapeDtypeStruct((N, 128), F32),
        jax.ShapeDtypeStruct((8, N), F32),
    )
    return pl.pallas_call(_proj_body, out_shape=outs)(x, Md, Ms, Ma)


def _merge_body(dp, recip, csum):
    d = jnp.sum(dp[...], axis=0)
    rec = 1.0 / (d + 1e-16)
    recip[...] = rec
    csum[...] = d * rec


def _merge(dp, nh):
    outs = (
        jax.ShapeDtypeStruct((nh, N), F32),
        jax.ShapeDtypeStruct((nh, N), F32),
    )
    return pl.pallas_call(_merge_body, out_shape=outs)(dp)


def _make_pass_a(nh):
    EW = ET // NW
    NV = EW // 16

    def body(src_h, dst_h, t0_h, t1_h, scalT_h, ra_h, s_out, dp_out,
             as_v, ad_v, dn_v, ra_v, src_c, dst_c, t0_c, t1_c, s_c,
             kbuf, vbuf):
        wid = lax.axis_index("s") * 2 + lax.axis_index("c")
        base = wid * EW
        pltpu.sync_copy(ra_h, ra_v)
        pltpu.sync_copy(src_h.at[pl.ds(base, EW)], src_c)
        pltpu.sync_copy(dst_h.at[pl.ds(base, EW)], dst_c)
        pltpu.sync_copy(t0_h.at[pl.ds(base, EW)], t0_c)
        pltpu.sync_copy(t1_h.at[pl.ds(base, EW)], t1_c)
        lane = lax.iota(I32, 16)
        zf = jnp.zeros((16,), F32)
        for h in range(nh):
            pltpu.sync_copy(scalT_h.at[pl.ds(h * N, N)], as_v)
            pltpu.sync_copy(scalT_h.at[pl.ds((nh + h) * N, N)], ad_v)

            def zbody(i, _):
                dn_v[pl.ds(i * 16, 16)] = zf
                return 0
            lax.fori_loop(0, N // 16, zbody, 0)

            def vbody(j, _):
                o = j * 16
                srcv = src_c[pl.ds(o, 16)]
                dstv = dst_c[pl.ds(o, 16)]
                t0v = t0_c[pl.ds(o, 16)]
                t1v = t1_c[pl.ds(o, 16)]
                av = (plsc.load_gather(as_v, [srcv])
                      + plsc.load_gather(ad_v, [dstv])
                      + plsc.load_gather(ra_v, [t0v * 4 + h])
                      + plsc.load_gather(ra_v, [t1v * 4 + h]))
                av = jnp.maximum(av, 0.2 * av)
                sv = jnp.exp(av)
                s_c[pl.ds(o, 16)] = sv
                sk, sv2 = plsc.sort_key_val(srcv, sv)
                kbuf[...] = sk
                vv = sv2
                for d in (1, 2, 4, 8):
                    idd = jnp.maximum(lane - d, 0)
                    kb = plsc.load_gather(kbuf, [idd])
                    vbuf[...] = vv
                    vb = plsc.load_gather(vbuf, [idd])
                    vv = vv + jnp.where((lane >= d) & (kb == sk), vb, 0.0)
                nxt = plsc.load_gather(kbuf, [jnp.minimum(lane + 1, 15)])
                is_last = (lane == 15) | (nxt != sk)
                plsc.addupdate_scatter(dn_v, [sk], vv, mask=is_last)
                return 0
            lax.fori_loop(0, NV, vbody, 0)
            pltpu.sync_copy(s_c, s_out.at[pl.ds(h * ET + base, EW)])
            pltpu.sync_copy(dn_v, dp_out.at[pl.ds((wid * nh + h) * N, N)])

    return functools.partial(
        pl.kernel, body,
        out_type=(jax.ShapeDtypeStruct((nh * ET,), F32),
                  jax.ShapeDtypeStruct((NW * nh * N,), F32)),
        scratch_types=[
            pltpu.VMEM((N,), F32),
            pltpu.VMEM((N,), F32),
            pltpu.VMEM((N,), F32),
            pltpu.VMEM((TAB * 4,), F32),
            pltpu.VMEM((EW,), I32),
            pltpu.VMEM((EW,), I32),
            pltpu.VMEM((EW,), I32),
            pltpu.VMEM((EW,), I32),
            pltpu.VMEM((EW,), F32),
            pltpu.VMEM((16,), I32),
            pltpu.VMEM((16,), F32),
        ],
        **_SC_PARAMS)()


def _make_pass_b(nh):
    EW = ET // 2
    C = 640
    NCH = EW // C
    NVC = C * 8 // 16

    def body(src_h, dst_h, t0_h, t1_h, s_h, recip_h, xd2_h, tabG_h, zero_h,
             acc_out, acc_v, recip_v, tab_v, src_c, dst_c, t0_c, t1_c, s_c,
             idx_v, rows_v, sem):
        wid = lax.axis_index("s") * 2 + lax.axis_index("c")
        g = wid // 2
        half = wid - g * 2
        h = (g * nh) // 16
        pltpu.sync_copy(recip_h.at[pl.ds(h * N, N)], recip_v)
        pltpu.sync_copy(tabG_h.at[pl.ds(g * TAB * 8, TAB * 8)], tab_v)
        pltpu.sync_copy(zero_h, acc_v)
        lane = lax.iota(I32, 16)
        c01 = lane >> 3
        col = jnp.where(lane < 8, lane, 15 - lane)
        ebase = half * EW

        def cbody(ci, _):
            cb = ebase + ci * C
            pltpu.sync_copy(src_h.at[pl.ds(cb, C)], src_c)
            pltpu.sync_copy(dst_h.at[pl.ds(cb, C)], dst_c)
            pltpu.sync_copy(t0_h.at[pl.ds(cb, C)], t0_c)
            pltpu.sync_copy(t1_h.at[pl.ds(cb, C)], t1_c)
            pltpu.sync_copy(s_h.at[pl.ds(h * ET + cb, C)], s_c)

            def ibody(j, _):
                dstv = dst_c[pl.ds(j * 16, 16)]
                row = j // 8
                off = (j - row * 8) * 16
                idx_v[row, pl.ds(off, 16)] = dstv * 16 + g
                return 0
            lax.fori_loop(0, C // 16, ibody, 0)
            cps = [pltpu.async_copy(xd2_h.at[idx_v.at[k]],
                                    rows_v.at[pl.ds(k * 128, 128)], sem)
                   for k in range(C // 128)]
            for cp in cps:
                cp.wait()

            def vbody(j, _):
                esp = c01 + j * 2
                srcp = plsc.load_gather(src_c, [esp])
                sv = plsc.load_gather(s_c, [esp])
                rv = plsc.load_gather(recip_v, [srcp])
                coef = sv * rv
                t0p = plsc.load_gather(t0_c, [esp])
                t1p = plsc.load_gather(t1_c, [esp])
                tr0 = plsc.load_gather(tab_v, [t0p * 8 + col])
                tr1 = plsc.load_gather(tab_v, [t1p * 8 + col])
                row = plsc.load_gather(rows_v, [esp, col])
                val = (row + tr0 + tr1) * coef
                osrc = lax.rev(srcp, (0,))
                oval = lax.rev(val, (0,))
                isdup = srcp == osrc
                val = jnp.where(isdup, val + oval, val)
                keep = jnp.logical_or(~isdup, lane < 8)
                plsc.addupdate_scatter(acc_v, [srcp, col], val, mask=keep)
                return 0
            lax.fori_loop(0, NVC, vbody, 0)
            return 0
        lax.fori_loop(0, NCH, cbody, 0)
        pltpu.sync_copy(acc_v, acc_out.at[g, half])

    return functools.partial(
        pl.kernel, body,
        out_type=jax.ShapeDtypeStruct((16, 2, N, 8), F32),
        scratch_types=[
            pltpu.VMEM((N, 8), F32),
            pltpu.VMEM((N,), F32),
            pltpu.VMEM((TAB * 8,), F32),
            pltpu.VMEM((C,), I32),
            pltpu.VMEM((C,), I32),
            pltpu.VMEM((C,), I32),
            pltpu.VMEM((C,), I32),
            pltpu.VMEM((C,), F32),
            pltpu.VMEM((C // 128, 128), I32),
            pltpu.VMEM((C, 8), F32),
            pltpu.SemaphoreType.DMA,
        ],
        **_SC_PARAMS)()


def _make_pass_c(nh):
    R = 400
    NWC = N // R

    def body(accp_h, xs_h, csum_h, out_h, ab0, ab1, obuf, xsb, csb):
        wid = lax.axis_index("s") * 2 + lax.axis_index("c")

        @pl.when(wid < NWC)
        def _():
            base = wid * R
            lane = lax.iota(I32, 16)
            c01 = lane >> 3
            col = jnp.where(lane < 8, lane, 15 - lane)
            for h in range(nh):
                pltpu.sync_copy(csum_h.at[pl.ds(h * N + base, R)],
                                csb.at[pl.ds(h * R, R)])
            for g in range(16):
                pltpu.sync_copy(
                    accp_h.at[pl.ds(((g * 2) * N + base) * 8, R * 8)], ab0)
                pltpu.sync_copy(
                    accp_h.at[pl.ds(((g * 2 + 1) * N + base) * 8, R * 8)], ab1)

                def rbody(j, _):
                    esp = c01 + j * 2
                    v = (plsc.load_gather(ab0, [esp * 8 + col])
                         + plsc.load_gather(ab1, [esp * 8 + col]))
                    plsc.store_scatter(obuf, [esp, g * 8 + col], v)
                    return 0
                lax.fori_loop(0, R * 8 // 16, rbody, 0)
            pltpu.sync_copy(xs_h.at[pl.ds(base * 128, R * 128)], xsb)

            def ebody(r, _):
                for k in range(8):
                    h = k // 2 if nh == 4 else 0
                    cs = csb[pl.ds(h * R + r, 16)][0]
                    v = obuf[r, pl.ds(k * 16, 16)]
                    xv = xsb[pl.ds(r * 128 + k * 16, 16)]
                    v = v + xv * cs
                    v = jnp.where(v > 0.0, v, jnp.exp(v) - 1.0)
                    obuf[r, pl.ds(k * 16, 16)] = v
                return 0
            lax.fori_loop(0, R, ebody, 0)
            pltpu.sync_copy(obuf, out_h.at[pl.ds(base, R)])

    return functools.partial(
        pl.kernel, body,
        out_type=jax.ShapeDtypeStruct((N, 128), F32),
        scratch_types=[
            pltpu.VMEM((R * 8,), F32),
            pltpu.VMEM((R * 8,), F32),
            pltpu.VMEM((R, 128), F32),
            pltpu.VMEM((R * 128,), F32),
            pltpu.VMEM((4 * R + 16,), F32),
        ],
        **_SC_PARAMS)()


_PASS_A = {nh: _make_pass_a(nh) for nh in (4, 1)}
_PASS_B = {nh: _make_pass_b(nh) for nh in (4, 1)}
_PASS_C = {nh: _make_pass_c(nh) for nh in (4, 1)}


def _layer(src, dst, t0, t1, x, Md, Ms, Ma, tab, ra, zero8, nh):
    xd, xs, scalT = _proj(x, Md, Ms, Ma)
    s, dp = _PASS_A[nh](src, dst, t0, t1, scalT.reshape(8 * N),
                        ra.reshape(TAB * 4))
    recip, csum = _merge(dp.reshape(NW, nh, N), nh)
    xd2 = xd.reshape(N * 16, 8)
    tabG = tab.reshape(TAB, 16, 8).transpose(1, 0, 2).reshape(16 * TAB * 8)
    accp = _PASS_B[nh](src, dst, t0, t1, s, recip.reshape(nh * N), xd2, tabG,
                       zero8)
    return _PASS_C[nh](accp.reshape(16 * 2 * N * 8), xs.reshape(N * 128),
                       csum.reshape(nh * N))


def kernel(edge_index, x, r, edge_type, edge_type_nhop,
           W0, a0, W1, a1, W2, a2, W3, a3, W_out, a_out, W_rel):
    edge_index = edge_index.astype(I32)
    src = edge_index[0]
    dst = edge_index[1]
    t0 = jnp.concatenate([edge_type.astype(I32),
                          edge_type_nhop[:, 0].astype(I32)])
    t1 = jnp.concatenate([jnp.full((E1,), NREL, I32),
                          edge_type_nhop[:, 1].astype(I32)])

    (Md1, Ms1, Ma1, tab1, ra1, Ma2, tab2, ra2, r2) = _prep(
        W0, a0, W1, a1, W2, a2, W3, a3, W_out, a_out, W_rel, r)

    zero8 = jnp.zeros((N, 8), F32)
    h = _layer(src, dst, t0, t1, x, Md1, Ms1, Ma1, tab1, ra1, zero8, 4)
    Ms2 = W_out[0:128]
    Md2 = W_out[128:256]
    out = _layer(src, dst, t0, t1, h, Md2, Ms2, Ma2, tab2, ra2, zero8, 1)
    return (out, r2)


# speedup vs baseline: 8.5904x; 1.5253x over previous
import functools

import jax
import jax.numpy as jnp
from jax import lax
from jax.experimental import pallas as pl
from jax.experimental.pallas import tpu as pltpu
from jax.experimental.pallas import tpu_sc as plsc

N = 10000
E1 = 256000
EN = 64000
ET = E1 + EN
NREL = 200
TAB = 208
NW = 32
F32 = jnp.float32
I32 = jnp.int32

_SC_PARAMS = dict(
    mesh=plsc.VectorSubcoreMesh(core_axis_name="c", subcore_axis_name="s"),
    compiler_params=pltpu.CompilerParams(needs_layout_passes=False,
                                         use_tc_tiling_on_sc=False),
)


def _prep_body(W0, a0, W1, a1, W2, a2, W3, a3, W_out, a_out, W_rel, r,
               Md1, Ms1, Ma1, tab1, ra1, Ma2, tab2, ra2, r2):
    Ws = [W0[...], W1[...], W2[...], W3[...]]
    As = [a0[...], a1[...], a2[...], a3[...]]
    Md1[...] = jnp.concatenate([w[128:256] for w in Ws], axis=1)
    Ms1[...] = jnp.concatenate([w[0:128] for w in Ws], axis=1)
    Ma1[...] = jnp.concatenate(
        [jnp.dot(w[0:128], a, preferred_element_type=F32) for w, a in zip(Ws, As)]
        + [jnp.dot(w[128:256], a, preferred_element_type=F32) for w, a in zip(Ws, As)],
        axis=1)
    rr = r[...]
    wec = jnp.concatenate([w[256:272] for w in Ws], axis=1)
    t1v = jnp.dot(rr, wec, preferred_element_type=F32)
    z8 = jnp.zeros((8, 128), F32)
    tab1[...] = jnp.concatenate([t1v, z8], axis=0)
    ra1v = jnp.concatenate(
        [jnp.dot(t1v[:, 32 * k:32 * k + 32], As[k], preferred_element_type=F32)
         for k in range(4)], axis=1)
    ra1[...] = jnp.concatenate([ra1v, jnp.zeros((8, 4), F32)], axis=0)
    wo = W_out[...]
    ao = a_out[...]
    Ma2[...] = jnp.concatenate(
        [jnp.dot(wo[0:128], ao, preferred_element_type=F32),
         jnp.dot(wo[128:256], ao, preferred_element_type=F32),
         jnp.zeros((128, 6), F32)], axis=1)
    r2v = jnp.dot(rr, W_rel[...], preferred_element_type=F32)
    r2[...] = r2v
    t2v = jnp.dot(r2v, wo[256:384], preferred_element_type=F32)
    tab2[...] = jnp.concatenate([t2v, z8], axis=0)
    ra2v = jnp.dot(t2v, ao, preferred_element_type=F32)
    ra2[...] = jnp.concatenate(
        [jnp.concatenate([ra2v, jnp.zeros((200, 3), F32)], axis=1),
         jnp.zeros((8, 4), F32)], axis=0)


def _prep(W0, a0, W1, a1, W2, a2, W3, a3, W_out, a_out, W_rel, r):
    outs = (
        jax.ShapeDtypeStruct((128, 128), F32),
        jax.ShapeDtypeStruct((128, 128), F32),
        jax.ShapeDtypeStruct((128, 8), F32),
        jax.ShapeDtypeStruct((TAB, 128), F32),
        jax.ShapeDtypeStruct((TAB, 4), F32),
        jax.ShapeDtypeStruct((128, 8), F32),
        jax.ShapeDtypeStruct((TAB, 128), F32),
        jax.ShapeDtypeStruct((TAB, 4), F32),
        jax.ShapeDtypeStruct((200, 128), F32),
    )
    return pl.pallas_call(_prep_body, out_shape=outs)(
        W0, a0, W1, a1, W2, a2, W3, a3, W_out, a_out, W_rel, r)


def _proj_body(x, Md, Ms, Ma, xd, xs, scalT):
    xv = x[...]
    xd[...] = jnp.dot(xv, Md[...], preferred_element_type=F32)
    xs[...] = jnp.dot(xv, Ms[...], preferred_element_type=F32)
    scalT[...] = lax.dot_general(Ma[...], xv, (((0,), (1,)), ((), ())),
                                 preferred_element_type=F32)


def _proj(x, Md, Ms, Ma):
    outs = (
        jax.ShapeDtypeStruct((N, 128), F32),
        jax.ShapeDtypeStruct((N, 128), F32),
        jax.ShapeDtypeStruct((8, N), F32),
    )
    return pl.pallas_call(_proj_body, out_shape=outs)(x, Md, Ms, Ma)


def _merge_body(dp, recip, csum):
    d = jnp.sum(dp[...], axis=0)
    rec = 1.0 / (d + 1e-16)
    recip[...] = rec
    csum[...] = d * rec


def _merge(dp, nh):
    outs = (
        jax.ShapeDtypeStruct((nh, N), F32),
        jax.ShapeDtypeStruct((nh, N), F32),
    )
    return pl.pallas_call(_merge_body, out_shape=outs)(dp)


def _make_pass_a(nh):
    EW = ET // NW
    NV = EW // 16

    def body(src_h, dst_h, t0_h, t1_h, scalT_h, ra_h, s_out, dp_out,
             as_v, ad_v, dn_v, ra_v, src_c, dst_c, t0_c, t1_c, s_c,
             kbuf, vbuf):
        wid = lax.axis_index("s") * 2 + lax.axis_index("c")
        base = wid * EW
        pltpu.sync_copy(ra_h, ra_v)
        pltpu.sync_copy(src_h.at[pl.ds(base, EW)], src_c)
        pltpu.sync_copy(dst_h.at[pl.ds(base, EW)], dst_c)
        pltpu.sync_copy(t0_h.at[pl.ds(base, EW)], t0_c)
        pltpu.sync_copy(t1_h.at[pl.ds(base, EW)], t1_c)
        lane = lax.iota(I32, 16)
        zf = jnp.zeros((16,), F32)
        for h in range(nh):
            pltpu.sync_copy(scalT_h.at[pl.ds(h * N, N)], as_v)
            pltpu.sync_copy(scalT_h.at[pl.ds((nh + h) * N, N)], ad_v)

            def zbody(i, _):
                dn_v[pl.ds(i * 16, 16)] = zf
                return 0
            lax.fori_loop(0, N // 16, zbody, 0)

            def vbody(j, _):
                o = j * 16
                srcv = src_c[pl.ds(o, 16)]
                dstv = dst_c[pl.ds(o, 16)]
                t0v = t0_c[pl.ds(o, 16)]
                t1v = t1_c[pl.ds(o, 16)]
                av = (plsc.load_gather(as_v, [srcv])
                      + plsc.load_gather(ad_v, [dstv])
                      + plsc.load_gather(ra_v, [t0v * 4 + h])
                      + plsc.load_gather(ra_v, [t1v * 4 + h]))
                av = jnp.maximum(av, 0.2 * av)
                sv = jnp.exp(av)
                s_c[pl.ds(o, 16)] = sv
                sk, sv2 = plsc.sort_key_val(srcv, sv)
                kbuf[...] = sk
                vv = sv2
                for d in (1, 2, 4, 8):
                    idd = jnp.maximum(lane - d, 0)
                    kb = plsc.load_gather(kbuf, [idd])
                    vbuf[...] = vv
                    vb = plsc.load_gather(vbuf, [idd])
                    vv = vv + jnp.where((lane >= d) & (kb == sk), vb, 0.0)
                nxt = plsc.load_gather(kbuf, [jnp.minimum(lane + 1, 15)])
                is_last = (lane == 15) | (nxt != sk)
                plsc.addupdate_scatter(dn_v, [sk], vv, mask=is_last)
                return 0
            lax.fori_loop(0, NV, vbody, 0)
            pltpu.sync_copy(s_c, s_out.at[pl.ds(h * ET + base, EW)])
            pltpu.sync_copy(dn_v, dp_out.at[pl.ds((wid * nh + h) * N, N)])

    return functools.partial(
        pl.kernel, body,
        out_type=(jax.ShapeDtypeStruct((nh * ET,), F32),
                  jax.ShapeDtypeStruct((NW * nh * N,), F32)),
        scratch_types=[
            pltpu.VMEM((N,), F32),
            pltpu.VMEM((N,), F32),
            pltpu.VMEM((N,), F32),
            pltpu.VMEM((TAB * 4,), F32),
            pltpu.VMEM((EW,), I32),
            pltpu.VMEM((EW,), I32),
            pltpu.VMEM((EW,), I32),
            pltpu.VMEM((EW,), I32),
            pltpu.VMEM((EW,), F32),
            pltpu.VMEM((16,), I32),
            pltpu.VMEM((16,), F32),
        ],
        **_SC_PARAMS)()


def _make_pass_b(nh):
    EW = ET // 2
    C = 640
    NG = C // 128
    NCH = EW // C
    NVC = C * 8 // 16

    def body(sd_h, t01_h, s_h, recip_h, xd2_h, tabG_h, zero_h,
             acc_out, acc_v, recip_v, tab_v, sd_c, t01_c, s_c,
             idx_v, rows_v, msem, gsem):
        wid = lax.axis_index("s") * 2 + lax.axis_index("c")
        g = wid // 2
        half = wid - g * 2
        h = (g * nh) // 16
        pltpu.sync_copy(recip_h.at[pl.ds(h * N, N)], recip_v)
        pltpu.sync_copy(tabG_h.at[pl.ds(g * TAB * 8, TAB * 8)], tab_v)
        pltpu.sync_copy(zero_h, acc_v)
        lane = lax.iota(I32, 16)
        c01 = lane >> 3
        col = jnp.where(lane < 8, lane, 15 - lane)
        lane2p1 = lane * 2 + 1
        ebase = half * EW

        def _slot(c):
            return c - (c // 3) * 3

        def _par(c):
            return c - (c // 2) * 2

        def _meta_copies(c):
            cb = ebase + c * C
            slot, par = _slot(c), _par(c)
            return [
                pltpu.make_async_copy(sd_h.at[pl.ds(cb * 2, 2 * C)],
                                      sd_c.at[slot], msem.at[par]),
                pltpu.make_async_copy(t01_h.at[pl.ds(cb, C)],
                                      t01_c.at[slot], msem.at[par]),
                pltpu.make_async_copy(s_h.at[pl.ds(h * ET + cb, C)],
                                      s_c.at[slot], msem.at[par]),
            ]

        def meta_fire(c):
            for cp in _meta_copies(c):
                cp.start()

        def meta_wait(c):
            for cp in _meta_copies(c):
                cp.wait()

        def _gather_copies(c):
            par = _par(c)
            return [
                pltpu.make_async_copy(xd2_h.at[idx_v.at[par, k]],
                                      rows_v.at[par, pl.ds(k * 128, 128)],
                                      gsem.at[par])
                for k in range(NG)
            ]

        def gather_fire(c):
            slot, par = _slot(c), _par(c)
            ms = jnp.full((16,), slot, I32)

            def ibody(j, _):
                dstv = plsc.load_gather(sd_c, [ms, j * 32 + lane2p1])
                row = j // 8
                off = (j - row * 8) * 16
                idx_v[par, row, pl.ds(off, 16)] = dstv * 16 + g
                return 0
            lax.fori_loop(0, C // 16, ibody, 0)
            for cp in _gather_copies(c):
                cp.start()

        def gather_wait(c):
            for cp in _gather_copies(c):
                cp.wait()

        meta_fire(0)
        meta_wait(0)
        gather_fire(0)
        meta_fire(1)

        def cbody(ci, _):
            @pl.when(ci + 1 < NCH)
            def _():
                meta_wait(ci + 1)
                gather_fire(ci + 1)

            @pl.when(ci + 2 < NCH)
            def _():
                meta_fire(ci + 2)

            gather_wait(ci)
            slot, par = _slot(ci), _par(ci)
            ms = jnp.full((16,), slot, I32)
            ps = jnp.full((16,), par, I32)

            def vbody(j, _):
                esp = c01 + j * 2
                srcp = plsc.load_gather(sd_c, [ms, esp * 2])
                sv = plsc.load_gather(s_c, [ms, esp])
                rv = plsc.load_gather(recip_v, [srcp])
                coef = sv * rv
                t01p = plsc.load_gather(t01_c, [ms, esp])
                tr0 = plsc.load_gather(tab_v, [(t01p >> 8) * 8 + col])
                tr1 = plsc.load_gather(tab_v, [(t01p & 255) * 8 + col])
                row = plsc.load_gather(rows_v, [ps, esp, col])
                val = (row + tr0 + tr1) * coef
                osrc = lax.rev(srcp, (0,))
                oval = lax.rev(val, (0,))
                isdup = srcp == osrc
                val = jnp.where(isdup, val + oval, val)
                keep = jnp.logical_or(~isdup, lane < 8)
                plsc.addupdate_scatter(acc_v, [srcp, col], val, mask=keep)
                return 0
            lax.fori_loop(0, NVC, vbody, 0)
            return 0
        lax.fori_loop(0, NCH, cbody, 0)
        pltpu.sync_copy(acc_v, acc_out.at[g, half])

    return functools.partial(
        pl.kernel, body,
        out_type=jax.ShapeDtypeStruct((16, 2, N, 8), F32),
        scratch_types=[
            pltpu.VMEM((N, 8), F32),
            pltpu.VMEM((N,), F32),
            pltpu.VMEM((TAB * 8,), F32),
            pltpu.VMEM((3, 2 * C), I32),
            pltpu.VMEM((3, C), I32),
            pltpu.VMEM((3, C), F32),
            pltpu.VMEM((2, NG, 128), I32),
            pltpu.VMEM((2, C, 8), F32),
            pltpu.SemaphoreType.DMA((2,)),
            pltpu.SemaphoreType.DMA((2,)),
        ],
        **_SC_PARAMS)()


def _make_pass_c(nh):
    R = 400
    NWC = N // R

    def body(accp_h, xs_h, csum_h, out_h, ab0, ab1, obuf, xsb, csb):
        wid = lax.axis_index("s") * 2 + lax.axis_index("c")

        @pl.when(wid < NWC)
        def _():
            base = wid * R
            lane = lax.iota(I32, 16)
            c01 = lane >> 3
            col = jnp.where(lane < 8, lane, 15 - lane)
            for h in range(nh):
                pltpu.sync_copy(csum_h.at[pl.ds(h * N + base, R)],
                                csb.at[pl.ds(h * R, R)])
            for g in range(16):
                pltpu.sync_copy(
                    accp_h.at[pl.ds(((g * 2) * N + base) * 8, R * 8)], ab0)
                pltpu.sync_copy(
                    accp_h.at[pl.ds(((g * 2 + 1) * N + base) * 8, R * 8)], ab1)

                def rbody(j, _):
                    esp = c01 + j * 2
                    v = (plsc.load_gather(ab0, [esp * 8 + col])
                         + plsc.load_gather(ab1, [esp * 8 + col]))
                    plsc.store_scatter(obuf, [esp, g * 8 + col], v)
                    return 0
                lax.fori_loop(0, R * 8 // 16, rbody, 0)
            pltpu.sync_copy(xs_h.at[pl.ds(base * 128, R * 128)], xsb)

            def ebody(r, _):
                for k in range(8):
                    h = k // 2 if nh == 4 else 0
                    cs = csb[pl.ds(h * R + r, 16)][0]
                    v = obuf[r, pl.ds(k * 16, 16)]
                    xv = xsb[pl.ds(r * 128 + k * 16, 16)]
                    v = v + xv * cs
                    v = jnp.where(v > 0.0, v, jnp.exp(v) - 1.0)
                    obuf[r, pl.ds(k * 16, 16)] = v
                return 0
            lax.fori_loop(0, R, ebody, 0)
            pltpu.sync_copy(obuf, out_h.at[pl.ds(base, R)])

    return functools.partial(
        pl.kernel, body,
        out_type=jax.ShapeDtypeStruct((N, 128), F32),
        scratch_types=[
            pltpu.VMEM((R * 8,), F32),
            pltpu.VMEM((R * 8,), F32),
            pltpu.VMEM((R, 128), F32),
            pltpu.VMEM((R * 128,), F32),
            pltpu.VMEM((4 * R + 16,), F32),
        ],
        **_SC_PARAMS)()


_PASS_A = {nh: _make_pass_a(nh) for nh in (4, 1)}
_PASS_B = {nh: _make_pass_b(nh) for nh in (4, 1)}
_PASS_C = {nh: _make_pass_c(nh) for nh in (4, 1)}


def _layer(src, dst, t0, t1, sd, t01, x, Md, Ms, Ma, tab, ra, zero8, nh):
    xd, xs, scalT = _proj(x, Md, Ms, Ma)
    s, dp = _PASS_A[nh](src, dst, t0, t1, scalT.reshape(8 * N),
                        ra.reshape(TAB * 4))
    recip, csum = _merge(dp.reshape(NW, nh, N), nh)
    xd2 = xd.reshape(N * 16, 8)
    tabG = tab.reshape(TAB, 16, 8).transpose(1, 0, 2).reshape(16 * TAB * 8)
    accp = _PASS_B[nh](sd, t01, s, recip.reshape(nh * N), xd2, tabG, zero8)
    return _PASS_C[nh](accp.reshape(16 * 2 * N * 8), xs.reshape(N * 128),
                       csum.reshape(nh * N))


def kernel(edge_index, x, r, edge_type, edge_type_nhop,
           W0, a0, W1, a1, W2, a2, W3, a3, W_out, a_out, W_rel):
    edge_index = edge_index.astype(I32)
    src = edge_index[0]
    dst = edge_index[1]
    t0 = jnp.concatenate([edge_type.astype(I32),
                          edge_type_nhop[:, 0].astype(I32)])
    t1 = jnp.concatenate([jnp.full((E1,), NREL, I32),
                          edge_type_nhop[:, 1].astype(I32)])
    sd = jnp.stack([src, dst], axis=1).reshape(2 * ET)
    t01 = t0 * 256 + t1

    (Md1, Ms1, Ma1, tab1, ra1, Ma2, tab2, ra2, r2) = _prep(
        W0, a0, W1, a1, W2, a2, W3, a3, W_out, a_out, W_rel, r)

    zero8 = jnp.zeros((N, 8), F32)
    h = _layer(src, dst, t0, t1, sd, t01, x, Md1, Ms1, Ma1, tab1, ra1,
               zero8, 4)
    Ms2 = W_out[0:128]
    Md2 = W_out[128:256]
    out = _layer(src, dst, t0, t1, sd, t01, h, Md2, Ms2, Ma2, tab2, ra2,
                 zero8, 1)
    return (out, r2)


# speedup vs baseline: 20.5050x; 2.3870x over previous
import functools

import jax
import jax.numpy as jnp
from jax import lax
from jax.experimental import pallas as pl
from jax.experimental.pallas import tpu as pltpu
from jax.experimental.pallas import tpu_sc as plsc

N = 10000
E1 = 256000
EN = 64000
ET = E1 + EN
NREL = 200
TAB = 208
NW = 32
F32 = jnp.float32
I32 = jnp.int32

_SC_PARAMS = dict(
    mesh=plsc.VectorSubcoreMesh(core_axis_name="c", subcore_axis_name="s"),
    compiler_params=pltpu.CompilerParams(needs_layout_passes=False,
                                         use_tc_tiling_on_sc=False),
)


def _prep_body(W0, a0, W1, a1, W2, a2, W3, a3, W_out, a_out, W_rel, r,
               Md1, Ms1, Ma1, tab1, ra1, Ma2, tab2, ra2, r2):
    Ws = [W0[...], W1[...], W2[...], W3[...]]
    As = [a0[...], a1[...], a2[...], a3[...]]
    Md1[...] = jnp.concatenate([w[128:256] for w in Ws], axis=1)
    Ms1[...] = jnp.concatenate([w[0:128] for w in Ws], axis=1)
    Ma1[...] = jnp.concatenate(
        [jnp.dot(w[0:128], a, preferred_element_type=F32) for w, a in zip(Ws, As)]
        + [jnp.dot(w[128:256], a, preferred_element_type=F32) for w, a in zip(Ws, As)],
        axis=1)
    rr = r[...]
    wec = jnp.concatenate([w[256:272] for w in Ws], axis=1)
    t1v = jnp.dot(rr, wec, preferred_element_type=F32)
    z8 = jnp.zeros((8, 128), F32)
    tab1[...] = jnp.concatenate([t1v, z8], axis=0)
    ra1v = jnp.concatenate(
        [jnp.dot(t1v[:, 32 * k:32 * k + 32], As[k], preferred_element_type=F32)
         for k in range(4)], axis=1)
    ra1[...] = jnp.concatenate([ra1v, jnp.zeros((8, 4), F32)], axis=0)
    wo = W_out[...]
    ao = a_out[...]
    Ma2[...] = jnp.concatenate(
        [jnp.dot(wo[0:128], ao, preferred_element_type=F32),
         jnp.dot(wo[128:256], ao, preferred_element_type=F32),
         jnp.zeros((128, 6), F32)], axis=1)
    r2v = jnp.dot(rr, W_rel[...], preferred_element_type=F32)
    r2[...] = r2v
    t2v = jnp.dot(r2v, wo[256:384], preferred_element_type=F32)
    tab2[...] = jnp.concatenate([t2v, z8], axis=0)
    ra2v = jnp.dot(t2v, ao, preferred_element_type=F32)
    ra2[...] = jnp.concatenate(
        [jnp.concatenate([ra2v, jnp.zeros((200, 3), F32)], axis=1),
         jnp.zeros((8, 4), F32)], axis=0)


def _prep(W0, a0, W1, a1, W2, a2, W3, a3, W_out, a_out, W_rel, r):
    outs = (
        jax.ShapeDtypeStruct((128, 128), F32),
        jax.ShapeDtypeStruct((128, 128), F32),
        jax.ShapeDtypeStruct((128, 8), F32),
        jax.ShapeDtypeStruct((TAB, 128), F32),
        jax.ShapeDtypeStruct((TAB, 4), F32),
        jax.ShapeDtypeStruct((128, 8), F32),
        jax.ShapeDtypeStruct((TAB, 128), F32),
        jax.ShapeDtypeStruct((TAB, 4), F32),
        jax.ShapeDtypeStruct((200, 128), F32),
    )
    return pl.pallas_call(_prep_body, out_shape=outs)(
        W0, a0, W1, a1, W2, a2, W3, a3, W_out, a_out, W_rel, r)


def _proj_body(x, Md, Ms, Ma, xd, xs, scalT):
    xv = x[...]
    xd[...] = jnp.dot(xv, Md[...], preferred_element_type=F32)
    xs[...] = jnp.dot(xv, Ms[...], preferred_element_type=F32)
    scalT[...] = lax.dot_general(Ma[...], xv, (((0,), (1,)), ((), ())),
                                 preferred_element_type=F32)


def _proj(x, Md, Ms, Ma):
    outs = (
        jax.ShapeDtypeStruct((N, 128), F32),
        jax.ShapeDtypeStruct((N, 128), F32),
        jax.ShapeDtypeStruct((8, N), F32),
    )
    return pl.pallas_call(_proj_body, out_shape=outs)(x, Md, Ms, Ma)


def _merge_body(dp, recip, csum):
    d = jnp.sum(dp[...], axis=0)
    rec = 1.0 / (d + 1e-16)
    recip[...] = rec
    csum[...] = d * rec


def _merge(dp, nh):
    outs = (
        jax.ShapeDtypeStruct((nh, N), F32),
        jax.ShapeDtypeStruct((nh, N), F32),
    )
    return pl.pallas_call(_merge_body, out_shape=outs)(dp)


def _make_pass_a(nh):
    EW = ET // NW
    NV = EW // 16

    def body(src_h, dst_h, t0_h, t1_h, scalT_h, ra_h, s_out, dp_out,
             as_v, ad_v, dn_v, ra_v, src_c, dst_c, t0_c, t1_c, s_c,
             kbuf, vbuf):
        wid = lax.axis_index("s") * 2 + lax.axis_index("c")
        base = wid * EW
        pltpu.sync_copy(ra_h, ra_v)
        pltpu.sync_copy(src_h.at[pl.ds(base, EW)], src_c)
        pltpu.sync_copy(dst_h.at[pl.ds(base, EW)], dst_c)
        pltpu.sync_copy(t0_h.at[pl.ds(base, EW)], t0_c)
        pltpu.sync_copy(t1_h.at[pl.ds(base, EW)], t1_c)
        lane = lax.iota(I32, 16)
        zf = jnp.zeros((16,), F32)
        for h in range(nh):
            pltpu.sync_copy(scalT_h.at[pl.ds(h * N, N)], as_v)
            pltpu.sync_copy(scalT_h.at[pl.ds((nh + h) * N, N)], ad_v)

            def zbody(i, _):
                dn_v[pl.ds(i * 16, 16)] = zf
                return 0
            lax.fori_loop(0, N // 16, zbody, 0)

            def vbody(j, _):
                o = j * 16
                srcv = src_c[pl.ds(o, 16)]
                dstv = dst_c[pl.ds(o, 16)]
                t0v = t0_c[pl.ds(o, 16)]
                t1v = t1_c[pl.ds(o, 16)]
                av = (plsc.load_gather(as_v, [srcv])
                      + plsc.load_gather(ad_v, [dstv])
                      + plsc.load_gather(ra_v, [t0v * 4 + h])
                      + plsc.load_gather(ra_v, [t1v * 4 + h]))
                av = jnp.maximum(av, 0.2 * av)
                sv = jnp.exp(av)
                s_c[pl.ds(o, 16)] = sv
                sk, sv2 = plsc.sort_key_val(srcv, sv)
                kbuf[...] = sk
                vv = sv2
                for d in (1, 2, 4, 8):
                    idd = jnp.maximum(lane - d, 0)
                    kb = plsc.load_gather(kbuf, [idd])
                    vbuf[...] = vv
                    vb = plsc.load_gather(vbuf, [idd])
                    vv = vv + jnp.where((lane >= d) & (kb == sk), vb, 0.0)
                nxt = plsc.load_gather(kbuf, [jnp.minimum(lane + 1, 15)])
                is_last = (lane == 15) | (nxt != sk)
                plsc.addupdate_scatter(dn_v, [sk], vv, mask=is_last)
                return 0
            lax.fori_loop(0, NV, vbody, 0)
            pltpu.sync_copy(s_c, s_out.at[pl.ds(h * ET + base, EW)])
            pltpu.sync_copy(dn_v, dp_out.at[pl.ds((wid * nh + h) * N, N)])

    return functools.partial(
        pl.kernel, body,
        out_type=(jax.ShapeDtypeStruct((nh * ET,), F32),
                  jax.ShapeDtypeStruct((NW * nh * N,), F32)),
        scratch_types=[
            pltpu.VMEM((N,), F32),
            pltpu.VMEM((N,), F32),
            pltpu.VMEM((N,), F32),
            pltpu.VMEM((TAB * 4,), F32),
            pltpu.VMEM((EW,), I32),
            pltpu.VMEM((EW,), I32),
            pltpu.VMEM((EW,), I32),
            pltpu.VMEM((EW,), I32),
            pltpu.VMEM((EW,), F32),
            pltpu.VMEM((16,), I32),
            pltpu.VMEM((16,), F32),
        ],
        **_SC_PARAMS)()


def _make_pass_b(nh):
    EW = ET // 2
    C = 640
    NG = C // 128
    NCH = EW // C
    NVC = C * 8 // 16

    def body(sd_h, t01_h, s_h, recip_h, xd2_h, tabG_h, zero_h,
             acc_out, acc_v, recip_v, tab_v, sd_c, t01_c, s_c,
             idx_v, rows_v, msem, gsem):
        wid = lax.axis_index("s") * 2 + lax.axis_index("c")
        g = wid // 2
        half = wid - g * 2
        h = (g * nh) // 16
        pltpu.sync_copy(recip_h.at[pl.ds(h * N, N)], recip_v)
        pltpu.sync_copy(tabG_h.at[pl.ds(g * TAB * 8, TAB * 8)], tab_v)
        pltpu.sync_copy(zero_h, acc_v)
        lane = lax.iota(I32, 16)
        c01 = lane >> 3
        col = jnp.where(lane < 8, lane, 15 - lane)
        lane2p1 = lane * 2 + 1
        ebase = half * EW

        def _slot(c):
            return c - (c // 3) * 3

        def _par(c):
            return c - (c // 2) * 2

        def _meta_copies(c):
            cb = ebase + c * C
            slot, par = _slot(c), _par(c)
            return [
                pltpu.make_async_copy(sd_h.at[pl.ds(cb * 2, 2 * C)],
                                      sd_c.at[slot], msem.at[par]),
                pltpu.make_async_copy(t01_h.at[pl.ds(cb, C)],
                                      t01_c.at[slot], msem.at[par]),
                pltpu.make_async_copy(s_h.at[pl.ds(h * ET + cb, C)],
                                      s_c.at[slot], msem.at[par]),
            ]

        def meta_fire(c):
            for cp in _meta_copies(c):
                cp.start()

        def meta_wait(c):
            for cp in _meta_copies(c):
                cp.wait()

        def _gather_copies(c):
            par = _par(c)
            return [
                pltpu.make_async_copy(xd2_h.at[idx_v.at[par, k]],
                                      rows_v.at[par, pl.ds(k * 128, 128)],
                                      gsem.at[par])
                for k in range(NG)
            ]

        def gather_fire(c):
            slot, par = _slot(c), _par(c)
            ms = jnp.full((16,), slot, I32)

            @plsc.parallel_loop(0, C // 16, unroll=2)
            def ibody(j):
                dstv = plsc.load_gather(sd_c, [ms, j * 32 + lane2p1])
                row = j // 8
                off = (j - row * 8) * 16
                idx_v[par, row, pl.ds(off, 16)] = dstv * 16 + g
            for cp in _gather_copies(c):
                cp.start()

        def gather_wait(c):
            for cp in _gather_copies(c):
                cp.wait()

        meta_fire(0)
        meta_wait(0)
        gather_fire(0)
        meta_fire(1)

        def cbody(ci, _):
            @pl.when(ci + 1 < NCH)
            def _():
                meta_wait(ci + 1)
                gather_fire(ci + 1)

            @pl.when(ci + 2 < NCH)
            def _():
                meta_fire(ci + 2)

            gather_wait(ci)
            slot, par = _slot(ci), _par(ci)
            ms = jnp.full((16,), slot, I32)
            ps = jnp.full((16,), par, I32)

            @plsc.parallel_loop(0, NVC, unroll=4)
            def vbody(j):
                esp = c01 + j * 2
                srcp = plsc.load_gather(sd_c, [ms, esp * 2])
                sv = plsc.load_gather(s_c, [ms, esp])
                rv = plsc.load_gather(recip_v, [srcp])
                coef = sv * rv
                t01p = plsc.load_gather(t01_c, [ms, esp])
                tr0 = plsc.load_gather(tab_v, [(t01p >> 8) * 8 + col])
                tr1 = plsc.load_gather(tab_v, [(t01p & 255) * 8 + col])
                row = plsc.load_gather(rows_v, [ps, esp, col])
                val = (row + tr0 + tr1) * coef
                osrc = lax.rev(srcp, (0,))
                oval = lax.rev(val, (0,))
                isdup = srcp == osrc
                val = jnp.where(isdup, val + oval, val)
                keep = jnp.logical_or(~isdup, lane < 8)
                plsc.addupdate_scatter(acc_v, [srcp, col], val, mask=keep)
            return 0
        lax.fori_loop(0, NCH, cbody, 0)
        pltpu.sync_copy(acc_v, acc_out.at[g, half])

    return functools.partial(
        pl.kernel, body,
        out_type=jax.ShapeDtypeStruct((16, 2, N, 8), F32),
        scratch_types=[
            pltpu.VMEM((N, 8), F32),
            pltpu.VMEM((N,), F32),
            pltpu.VMEM((TAB * 8,), F32),
            pltpu.VMEM((3, 2 * C), I32),
            pltpu.VMEM((3, C), I32),
            pltpu.VMEM((3, C), F32),
            pltpu.VMEM((2, NG, 128), I32),
            pltpu.VMEM((2, C, 8), F32),
            pltpu.SemaphoreType.DMA((2,)),
            pltpu.SemaphoreType.DMA((2,)),
        ],
        **_SC_PARAMS)()


def _make_pass_c(nh):
    R = 400
    NWC = N // R

    def body(accp_h, xs_h, csum_h, out_h, ab0, ab1, obuf, xsb, csb):
        wid = lax.axis_index("s") * 2 + lax.axis_index("c")

        @pl.when(wid < NWC)
        def _():
            base = wid * R
            lane = lax.iota(I32, 16)
            c01 = lane >> 3
            col = jnp.where(lane < 8, lane, 15 - lane)
            for h in range(nh):
                pltpu.sync_copy(csum_h.at[pl.ds(h * N + base, R)],
                                csb.at[pl.ds(h * R, R)])
            for g in range(16):
                pltpu.sync_copy(
                    accp_h.at[pl.ds(((g * 2) * N + base) * 8, R * 8)], ab0)
                pltpu.sync_copy(
                    accp_h.at[pl.ds(((g * 2 + 1) * N + base) * 8, R * 8)], ab1)

                @plsc.parallel_loop(0, R * 8 // 16, unroll=4)
                def rbody(j):
                    esp = c01 + j * 2
                    v = (plsc.load_gather(ab0, [esp * 8 + col])
                         + plsc.load_gather(ab1, [esp * 8 + col]))
                    plsc.store_scatter(obuf, [esp, g * 8 + col], v)
            pltpu.sync_copy(xs_h.at[pl.ds(base * 128, R * 128)], xsb)

            @plsc.parallel_loop(0, R, unroll=2)
            def ebody(r):
                for k in range(8):
                    h = k // 2 if nh == 4 else 0
                    cs = csb[pl.ds(h * R + r, 16)][0]
                    v = obuf[r, pl.ds(k * 16, 16)]
                    xv = xsb[pl.ds(r * 128 + k * 16, 16)]
                    v = v + xv * cs
                    v = jnp.where(v > 0.0, v, jnp.exp(v) - 1.0)
                    obuf[r, pl.ds(k * 16, 16)] = v
            pltpu.sync_copy(obuf, out_h.at[pl.ds(base, R)])

    return functools.partial(
        pl.kernel, body,
        out_type=jax.ShapeDtypeStruct((N, 128), F32),
        scratch_types=[
            pltpu.VMEM((R * 8,), F32),
            pltpu.VMEM((R * 8,), F32),
            pltpu.VMEM((R, 128), F32),
            pltpu.VMEM((R * 128,), F32),
            pltpu.VMEM((4 * R + 16,), F32),
        ],
        **_SC_PARAMS)()


_PASS_A = {nh: _make_pass_a(nh) for nh in (4, 1)}
_PASS_B = {nh: _make_pass_b(nh) for nh in (4, 1)}
_PASS_C = {nh: _make_pass_c(nh) for nh in (4, 1)}


def _layer(src, dst, t0, t1, sd, t01, x, Md, Ms, Ma, tab, ra, zero8, nh):
    xd, xs, scalT = _proj(x, Md, Ms, Ma)
    s, dp = _PASS_A[nh](src, dst, t0, t1, scalT.reshape(8 * N),
                        ra.reshape(TAB * 4))
    recip, csum = _merge(dp.reshape(NW, nh, N), nh)
    xd2 = xd.reshape(N * 16, 8)
    tabG = tab.reshape(TAB, 16, 8).transpose(1, 0, 2).reshape(16 * TAB * 8)
    accp = _PASS_B[nh](sd, t01, s, recip.reshape(nh * N), xd2, tabG, zero8)
    return _PASS_C[nh](accp.reshape(16 * 2 * N * 8), xs.reshape(N * 128),
                       csum.reshape(nh * N))


def kernel(edge_index, x, r, edge_type, edge_type_nhop,
           W0, a0, W1, a1, W2, a2, W3, a3, W_out, a_out, W_rel):
    edge_index = edge_index.astype(I32)
    src = edge_index[0]
    dst = edge_index[1]
    t0 = jnp.concatenate([edge_type.astype(I32),
                          edge_type_nhop[:, 0].astype(I32)])
    t1 = jnp.concatenate([jnp.full((E1,), NREL, I32),
                          edge_type_nhop[:, 1].astype(I32)])
    sd = jnp.stack([src, dst], axis=1).reshape(2 * ET)
    t01 = t0 * 256 + t1

    (Md1, Ms1, Ma1, tab1, ra1, Ma2, tab2, ra2, r2) = _prep(
        W0, a0, W1, a1, W2, a2, W3, a3, W_out, a_out, W_rel, r)

    zero8 = jnp.zeros((N, 8), F32)
    h = _layer(src, dst, t0, t1, sd, t01, x, Md1, Ms1, Ma1, tab1, ra1,
               zero8, 4)
    Ms2 = W_out[0:128]
    Md2 = W_out[128:256]
    out = _layer(src, dst, t0, t1, sd, t01, h, Md2, Ms2, Ma2, tab2, ra2,
                 zero8, 1)
    return (out, r2)


# speedup vs baseline: 23.6690x; 1.1543x over previous
import functools

import jax
import jax.numpy as jnp
from jax import lax
from jax.experimental import pallas as pl
from jax.experimental.pallas import tpu as pltpu
from jax.experimental.pallas import tpu_sc as plsc

N = 10000
E1 = 256000
EN = 64000
ET = E1 + EN
NREL = 200
TAB = 208
NW = 32
F32 = jnp.float32
I32 = jnp.int32

_SC_PARAMS = dict(
    mesh=plsc.VectorSubcoreMesh(core_axis_name="c", subcore_axis_name="s"),
    compiler_params=pltpu.CompilerParams(needs_layout_passes=False,
                                         use_tc_tiling_on_sc=False),
)


def _prep_body(W0, a0, W1, a1, W2, a2, W3, a3, W_out, a_out, W_rel, r,
               Md1, Ms1, Ma1, tab1, ra1, Ma2, tab2, ra2, r2):
    Ws = [W0[...], W1[...], W2[...], W3[...]]
    As = [a0[...], a1[...], a2[...], a3[...]]
    Md1[...] = jnp.concatenate([w[128:256] for w in Ws], axis=1)
    Ms1[...] = jnp.concatenate([w[0:128] for w in Ws], axis=1)
    Ma1[...] = jnp.concatenate(
        [jnp.dot(w[0:128], a, preferred_element_type=F32) for w, a in zip(Ws, As)]
        + [jnp.dot(w[128:256], a, preferred_element_type=F32) for w, a in zip(Ws, As)],
        axis=1)
    rr = r[...]
    wec = jnp.concatenate([w[256:272] for w in Ws], axis=1)
    t1v = jnp.dot(rr, wec, preferred_element_type=F32)
    z8 = jnp.zeros((8, 128), F32)
    tab1[...] = jnp.concatenate([t1v, z8], axis=0)
    ra1v = jnp.concatenate(
        [jnp.dot(t1v[:, 32 * k:32 * k + 32], As[k], preferred_element_type=F32)
         for k in range(4)], axis=1)
    ra1[...] = jnp.concatenate([ra1v, jnp.zeros((8, 4), F32)], axis=0)
    wo = W_out[...]
    ao = a_out[...]
    Ma2[...] = jnp.concatenate(
        [jnp.dot(wo[0:128], ao, preferred_element_type=F32),
         jnp.dot(wo[128:256], ao, preferred_element_type=F32),
         jnp.zeros((128, 6), F32)], axis=1)
    r2v = jnp.dot(rr, W_rel[...], preferred_element_type=F32)
    r2[...] = r2v
    t2v = jnp.dot(r2v, wo[256:384], preferred_element_type=F32)
    tab2[...] = jnp.concatenate([t2v, z8], axis=0)
    ra2v = jnp.dot(t2v, ao, preferred_element_type=F32)
    ra2[...] = jnp.concatenate(
        [jnp.concatenate([ra2v, jnp.zeros((200, 3), F32)], axis=1),
         jnp.zeros((8, 4), F32)], axis=0)


def _prep(W0, a0, W1, a1, W2, a2, W3, a3, W_out, a_out, W_rel, r):
    outs = (
        jax.ShapeDtypeStruct((128, 128), F32),
        jax.ShapeDtypeStruct((128, 128), F32),
        jax.ShapeDtypeStruct((128, 8), F32),
        jax.ShapeDtypeStruct((TAB, 128), F32),
        jax.ShapeDtypeStruct((TAB, 4), F32),
        jax.ShapeDtypeStruct((128, 8), F32),
        jax.ShapeDtypeStruct((TAB, 128), F32),
        jax.ShapeDtypeStruct((TAB, 4), F32),
        jax.ShapeDtypeStruct((200, 128), F32),
    )
    return pl.pallas_call(_prep_body, out_shape=outs)(
        W0, a0, W1, a1, W2, a2, W3, a3, W_out, a_out, W_rel, r)


def _proj_body(x, Md, Ms, Ma, xd, xs, scalT):
    xv = x[...]
    xd[...] = jnp.dot(xv, Md[...], preferred_element_type=F32)
    xs[...] = jnp.dot(xv, Ms[...], preferred_element_type=F32)
    scalT[...] = lax.dot_general(Ma[...], xv, (((0,), (1,)), ((), ())),
                                 preferred_element_type=F32)


def _proj(x, Md, Ms, Ma):
    outs = (
        jax.ShapeDtypeStruct((N, 128), F32),
        jax.ShapeDtypeStruct((N, 128), F32),
        jax.ShapeDtypeStruct((8, N), F32),
    )
    return pl.pallas_call(_proj_body, out_shape=outs)(x, Md, Ms, Ma)


def _merge_body(dp, recip, csum):
    d = jnp.sum(dp[...], axis=0)
    rec = 1.0 / (d + 1e-16)
    recip[...] = rec
    csum[...] = d * rec


def _merge(dp, nh):
    outs = (
        jax.ShapeDtypeStruct((nh, N), F32),
        jax.ShapeDtypeStruct((nh, N), F32),
    )
    return pl.pallas_call(_merge_body, out_shape=outs)(dp)


def _make_pass_a(nh):
    EW = ET // NW
    NV = EW // 16

    def body(src_h, dst_h, t0_h, t1_h, scalT_h, ra_h, s_out, dp_out,
             as_v, ad_v, dn_v, ra_v, src_c, dst_c, t0_c, t1_c, s_c):
        wid = lax.axis_index("s") * 2 + lax.axis_index("c")
        base = wid * EW
        pltpu.sync_copy(ra_h, ra_v)
        pltpu.sync_copy(src_h.at[pl.ds(base, EW)], src_c)
        pltpu.sync_copy(dst_h.at[pl.ds(base, EW)], dst_c)
        pltpu.sync_copy(t0_h.at[pl.ds(base, EW)], t0_c)
        pltpu.sync_copy(t1_h.at[pl.ds(base, EW)], t1_c)
        lane = lax.iota(I32, 16)
        zf = jnp.zeros((16,), F32)
        for h in range(nh):
            pltpu.sync_copy(scalT_h.at[pl.ds(h * N, N)], as_v)
            pltpu.sync_copy(scalT_h.at[pl.ds((nh + h) * N, N)], ad_v)

            def zbody(i, _):
                dn_v[pl.ds(i * 16, 16)] = zf
                return 0
            lax.fori_loop(0, N // 16, zbody, 0)

            @plsc.parallel_loop(0, NV, unroll=4)
            def vbody(j):
                o = j * 16
                srcv = src_c[pl.ds(o, 16)]
                dstv = dst_c[pl.ds(o, 16)]
                t0v = t0_c[pl.ds(o, 16)]
                t1v = t1_c[pl.ds(o, 16)]
                av = (plsc.load_gather(as_v, [srcv])
                      + plsc.load_gather(ad_v, [dstv])
                      + plsc.load_gather(ra_v, [t0v * 4 + h])
                      + plsc.load_gather(ra_v, [t1v * 4 + h]))
                av = jnp.maximum(av, 0.2 * av)
                sv = jnp.exp(av)
                s_c[pl.ds(o, 16)] = sv
                plsc.addupdate_scatter(dn_v, [srcv], sv)
            pltpu.sync_copy(s_c, s_out.at[pl.ds(h * ET + base, EW)])
            pltpu.sync_copy(dn_v, dp_out.at[pl.ds((wid * nh + h) * N, N)])

    return functools.partial(
        pl.kernel, body,
        out_type=(jax.ShapeDtypeStruct((nh * ET,), F32),
                  jax.ShapeDtypeStruct((NW * nh * N,), F32)),
        scratch_types=[
            pltpu.VMEM((N,), F32),
            pltpu.VMEM((N,), F32),
            pltpu.VMEM((N,), F32),
            pltpu.VMEM((TAB * 4,), F32),
            pltpu.VMEM((EW,), I32),
            pltpu.VMEM((EW,), I32),
            pltpu.VMEM((EW,), I32),
            pltpu.VMEM((EW,), I32),
            pltpu.VMEM((EW,), F32),
        ],
        **_SC_PARAMS)()


def _make_pass_b(nh):
    EW = ET // 2
    C = 640
    NG = C // 128
    NCH = EW // C
    NVC = C * 8 // 16

    def body(sd_h, t01_h, s_h, recip_h, xd2_h, tabG_h, zero_h,
             acc_out, acc_v, recip_v, tab_v, sd_c, t01_c, s_c,
             idx_v, rows_v, msem, gsem):
        wid = lax.axis_index("s") * 2 + lax.axis_index("c")
        g = wid // 2
        half = wid - g * 2
        h = (g * nh) // 16
        pltpu.sync_copy(recip_h.at[pl.ds(h * N, N)], recip_v)
        pltpu.sync_copy(tabG_h.at[pl.ds(g * TAB * 8, TAB * 8)], tab_v)
        pltpu.sync_copy(zero_h, acc_v)
        lane = lax.iota(I32, 16)
        c01 = lane >> 3
        col = jnp.where(lane < 8, lane, 15 - lane)
        lane2p1 = lane * 2 + 1
        ebase = half * EW

        def _slot(c):
            return c - (c // 3) * 3

        def _par(c):
            return c - (c // 2) * 2

        def _meta_copies(c):
            cb = ebase + c * C
            slot, par = _slot(c), _par(c)
            return [
                pltpu.make_async_copy(sd_h.at[pl.ds(cb * 2, 2 * C)],
                                      sd_c.at[slot], msem.at[par]),
                pltpu.make_async_copy(t01_h.at[pl.ds(cb, C)],
                                      t01_c.at[slot], msem.at[par]),
                pltpu.make_async_copy(s_h.at[pl.ds(h * ET + cb, C)],
                                      s_c.at[slot], msem.at[par]),
            ]

        def meta_fire(c):
            for cp in _meta_copies(c):
                cp.start()

        def meta_wait(c):
            for cp in _meta_copies(c):
                cp.wait()

        def _gather_copies(c):
            par = _par(c)
            return [
                pltpu.make_async_copy(xd2_h.at[idx_v.at[par, k]],
                                      rows_v.at[par, pl.ds(k * 128, 128)],
                                      gsem.at[par])
                for k in range(NG)
            ]

        def gather_fire(c):
            slot, par = _slot(c), _par(c)
            ms = jnp.full((16,), slot, I32)

            @plsc.parallel_loop(0, C // 16, unroll=2)
            def ibody(j):
                dstv = plsc.load_gather(sd_c, [ms, j * 32 + lane2p1])
                row = j // 8
                off = (j - row * 8) * 16
                idx_v[par, row, pl.ds(off, 16)] = dstv * 16 + g
            for cp in _gather_copies(c):
                cp.start()

        def gather_wait(c):
            for cp in _gather_copies(c):
                cp.wait()

        meta_fire(0)
        meta_wait(0)
        gather_fire(0)
        meta_fire(1)

        def cbody(ci, _):
            @pl.when(ci + 1 < NCH)
            def _():
                meta_wait(ci + 1)
                gather_fire(ci + 1)

            @pl.when(ci + 2 < NCH)
            def _():
                meta_fire(ci + 2)

            gather_wait(ci)
            slot, par = _slot(ci), _par(ci)
            ms = jnp.full((16,), slot, I32)
            ps = jnp.full((16,), par, I32)

            @plsc.parallel_loop(0, NVC, unroll=4)
            def vbody(j):
                esp = c01 + j * 2
                srcp = plsc.load_gather(sd_c, [ms, esp * 2])
                sv = plsc.load_gather(s_c, [ms, esp])
                rv = plsc.load_gather(recip_v, [srcp])
                coef = sv * rv
                t01p = plsc.load_gather(t01_c, [ms, esp])
                tr0 = plsc.load_gather(tab_v, [(t01p >> 8) * 8 + col])
                tr1 = plsc.load_gather(tab_v, [(t01p & 255) * 8 + col])
                row = plsc.load_gather(rows_v, [ps, esp, col])
                val = (row + tr0 + tr1) * coef
                plsc.addupdate_scatter(acc_v, [srcp, col], val)
            return 0
        lax.fori_loop(0, NCH, cbody, 0)
        pltpu.sync_copy(acc_v, acc_out.at[g, half])

    return functools.partial(
        pl.kernel, body,
        out_type=jax.ShapeDtypeStruct((16, 2, N, 8), F32),
        scratch_types=[
            pltpu.VMEM((N, 8), F32),
            pltpu.VMEM((N,), F32),
            pltpu.VMEM((TAB * 8,), F32),
            pltpu.VMEM((3, 2 * C), I32),
            pltpu.VMEM((3, C), I32),
            pltpu.VMEM((3, C), F32),
            pltpu.VMEM((2, NG, 128), I32),
            pltpu.VMEM((2, C, 8), F32),
            pltpu.SemaphoreType.DMA((2,)),
            pltpu.SemaphoreType.DMA((2,)),
        ],
        **_SC_PARAMS)()


def _make_pass_c(nh):
    R = 400
    NWC = N // R

    def body(accp_h, xs_h, csum_h, out_h, ab0, ab1, obuf, xsb, csb):
        wid = lax.axis_index("s") * 2 + lax.axis_index("c")

        @pl.when(wid < NWC)
        def _():
            base = wid * R
            lane = lax.iota(I32, 16)
            c01 = lane >> 3
            col = jnp.where(lane < 8, lane, 15 - lane)
            for h in range(nh):
                pltpu.sync_copy(csum_h.at[pl.ds(h * N + base, R)],
                                csb.at[pl.ds(h * R, R)])
            for g in range(16):
                pltpu.sync_copy(
                    accp_h.at[pl.ds(((g * 2) * N + base) * 8, R * 8)], ab0)
                pltpu.sync_copy(
                    accp_h.at[pl.ds(((g * 2 + 1) * N + base) * 8, R * 8)], ab1)

                @plsc.parallel_loop(0, R * 8 // 16, unroll=4)
                def rbody(j):
                    esp = c01 + j * 2
                    v = (plsc.load_gather(ab0, [esp * 8 + col])
                         + plsc.load_gather(ab1, [esp * 8 + col]))
                    plsc.store_scatter(obuf, [esp, g * 8 + col], v)
            pltpu.sync_copy(xs_h.at[pl.ds(base * 128, R * 128)], xsb)

            @plsc.parallel_loop(0, R, unroll=2)
            def ebody(r):
                for k in range(8):
                    h = k // 2 if nh == 4 else 0
                    cs = csb[pl.ds(h * R + r, 16)][0]
                    v = obuf[r, pl.ds(k * 16, 16)]
                    xv = xsb[pl.ds(r * 128 + k * 16, 16)]
                    v = v + xv * cs
                    v = jnp.where(v > 0.0, v, jnp.exp(v) - 1.0)
                    obuf[r, pl.ds(k * 16, 16)] = v
            pltpu.sync_copy(obuf, out_h.at[pl.ds(base, R)])

    return functools.partial(
        pl.kernel, body,
        out_type=jax.ShapeDtypeStruct((N, 128), F32),
        scratch_types=[
            pltpu.VMEM((R * 8,), F32),
            pltpu.VMEM((R * 8,), F32),
            pltpu.VMEM((R, 128), F32),
            pltpu.VMEM((R * 128,), F32),
            pltpu.VMEM((4 * R + 16,), F32),
        ],
        **_SC_PARAMS)()


_PASS_A = {nh: _make_pass_a(nh) for nh in (4, 1)}
_PASS_B = {nh: _make_pass_b(nh) for nh in (4, 1)}
_PASS_C = {nh: _make_pass_c(nh) for nh in (4, 1)}


def _layer(src, dst, t0, t1, sd, t01, x, Md, Ms, Ma, tab, ra, zero8, nh):
    xd, xs, scalT = _proj(x, Md, Ms, Ma)
    s, dp = _PASS_A[nh](src, dst, t0, t1, scalT.reshape(8 * N),
                        ra.reshape(TAB * 4))
    recip, csum = _merge(dp.reshape(NW, nh, N), nh)
    xd2 = xd.reshape(N * 16, 8)
    tabG = tab.reshape(TAB, 16, 8).transpose(1, 0, 2).reshape(16 * TAB * 8)
    accp = _PASS_B[nh](sd, t01, s, recip.reshape(nh * N), xd2, tabG, zero8)
    return _PASS_C[nh](accp.reshape(16 * 2 * N * 8), xs.reshape(N * 128),
                       csum.reshape(nh * N))


def kernel(edge_index, x, r, edge_type, edge_type_nhop,
           W0, a0, W1, a1, W2, a2, W3, a3, W_out, a_out, W_rel):
    edge_index = edge_index.astype(I32)
    src = edge_index[0]
    dst = edge_index[1]
    t0 = jnp.concatenate([edge_type.astype(I32),
                          edge_type_nhop[:, 0].astype(I32)])
    t1 = jnp.concatenate([jnp.full((E1,), NREL, I32),
                          edge_type_nhop[:, 1].astype(I32)])
    sd = jnp.stack([src, dst], axis=1).reshape(2 * ET)
    t01 = t0 * 256 + t1

    (Md1, Ms1, Ma1, tab1, ra1, Ma2, tab2, ra2, r2) = _prep(
        W0, a0, W1, a1, W2, a2, W3, a3, W_out, a_out, W_rel, r)

    zero8 = jnp.zeros((N, 8), F32)
    h = _layer(src, dst, t0, t1, sd, t01, x, Md1, Ms1, Ma1, tab1, ra1,
               zero8, 4)
    Ms2 = W_out[0:128]
    Md2 = W_out[128:256]
    out = _layer(src, dst, t0, t1, sd, t01, h, Md2, Ms2, Ma2, tab2, ra2,
                 zero8, 1)
    return (out, r2)


# speedup vs baseline: 25.6505x; 1.0837x over previous
import functools

import jax
import jax.numpy as jnp
from jax import lax
from jax.experimental import pallas as pl
from jax.experimental.pallas import tpu as pltpu
from jax.experimental.pallas import tpu_sc as plsc

N = 10000
E1 = 256000
EN = 64000
ET = E1 + EN
NREL = 200
TAB = 208
NW = 32
F32 = jnp.float32
I32 = jnp.int32

_SC_PARAMS = dict(
    mesh=plsc.VectorSubcoreMesh(core_axis_name="c", subcore_axis_name="s"),
    compiler_params=pltpu.CompilerParams(needs_layout_passes=False,
                                         use_tc_tiling_on_sc=False),
)


def _prep_body(W0, a0, W1, a1, W2, a2, W3, a3, W_out, a_out, W_rel, r,
               Md1, Ms1, Ma1, tab1, ra1, Ma2, tab2, ra2, r2):
    Ws = [W0[...], W1[...], W2[...], W3[...]]
    As = [a0[...], a1[...], a2[...], a3[...]]
    Md1[...] = jnp.concatenate([w[128:256] for w in Ws], axis=1)
    Ms1[...] = jnp.concatenate([w[0:128] for w in Ws], axis=1)
    Ma1[...] = jnp.concatenate(
        [jnp.dot(w[0:128], a, preferred_element_type=F32) for w, a in zip(Ws, As)]
        + [jnp.dot(w[128:256], a, preferred_element_type=F32) for w, a in zip(Ws, As)],
        axis=1)
    rr = r[...]
    wec = jnp.concatenate([w[256:272] for w in Ws], axis=1)
    t1v = jnp.dot(rr, wec, preferred_element_type=F32)
    z8 = jnp.zeros((8, 128), F32)
    tab1[...] = jnp.concatenate([t1v, z8], axis=0)
    ra1v = jnp.concatenate(
        [jnp.dot(t1v[:, 32 * k:32 * k + 32], As[k], preferred_element_type=F32)
         for k in range(4)], axis=1)
    ra1[...] = jnp.concatenate([ra1v, jnp.zeros((8, 4), F32)], axis=0)
    wo = W_out[...]
    ao = a_out[...]
    Ma2[...] = jnp.concatenate(
        [jnp.dot(wo[0:128], ao, preferred_element_type=F32),
         jnp.dot(wo[128:256], ao, preferred_element_type=F32),
         jnp.zeros((128, 6), F32)], axis=1)
    r2v = jnp.dot(rr, W_rel[...], preferred_element_type=F32)
    r2[...] = r2v
    t2v = jnp.dot(r2v, wo[256:384], preferred_element_type=F32)
    tab2[...] = jnp.concatenate([t2v, z8], axis=0)
    ra2v = jnp.dot(t2v, ao, preferred_element_type=F32)
    ra2[...] = jnp.concatenate(
        [jnp.concatenate([ra2v, jnp.zeros((200, 3), F32)], axis=1),
         jnp.zeros((8, 4), F32)], axis=0)


def _prep(W0, a0, W1, a1, W2, a2, W3, a3, W_out, a_out, W_rel, r):
    outs = (
        jax.ShapeDtypeStruct((128, 128), F32),
        jax.ShapeDtypeStruct((128, 128), F32),
        jax.ShapeDtypeStruct((128, 8), F32),
        jax.ShapeDtypeStruct((TAB, 128), F32),
        jax.ShapeDtypeStruct((TAB, 4), F32),
        jax.ShapeDtypeStruct((128, 8), F32),
        jax.ShapeDtypeStruct((TAB, 128), F32),
        jax.ShapeDtypeStruct((TAB, 4), F32),
        jax.ShapeDtypeStruct((200, 128), F32),
    )
    return pl.pallas_call(_prep_body, out_shape=outs)(
        W0, a0, W1, a1, W2, a2, W3, a3, W_out, a_out, W_rel, r)


def _proj_body(x, Md, Ms, Ma, xd, xs, scalT):
    xv = x[...]
    xd[...] = jnp.dot(xv, Md[...], preferred_element_type=F32)
    xs[...] = jnp.dot(xv, Ms[...], preferred_element_type=F32)
    scalT[...] = lax.dot_general(Ma[...], xv, (((0,), (1,)), ((), ())),
                                 preferred_element_type=F32)


def _proj(x, Md, Ms, Ma):
    outs = (
        jax.ShapeDtypeStruct((N, 128), F32),
        jax.ShapeDtypeStruct((N, 128), F32),
        jax.ShapeDtypeStruct((8, N), F32),
    )
    return pl.pallas_call(_proj_body, out_shape=outs)(x, Md, Ms, Ma)


def _merge_body(dp, recip, csum):
    d = jnp.sum(dp[...], axis=0)
    rec = 1.0 / (d + 1e-16)
    recip[...] = rec
    csum[...] = d * rec


def _merge(dp, nh):
    outs = (
        jax.ShapeDtypeStruct((nh, N), F32),
        jax.ShapeDtypeStruct((nh, N), F32),
    )
    return pl.pallas_call(_merge_body, out_shape=outs)(dp)


def _make_pass_a(nh):
    EW = ET // NW
    NV = EW // 16

    def body(src_h, dst_h, t0_h, t1_h, scalT_h, ra_h, s_out, dp_out,
             as_v, ad_v, dn_v, ra_v, src_c, dst_c, t0_c, t1_c, s_c):
        wid = lax.axis_index("s") * 2 + lax.axis_index("c")
        base = wid * EW
        pltpu.sync_copy(ra_h, ra_v)
        pltpu.sync_copy(src_h.at[pl.ds(base, EW)], src_c)
        pltpu.sync_copy(dst_h.at[pl.ds(base, EW)], dst_c)
        pltpu.sync_copy(t0_h.at[pl.ds(base, EW)], t0_c)
        pltpu.sync_copy(t1_h.at[pl.ds(base, EW)], t1_c)
        lane = lax.iota(I32, 16)
        zf = jnp.zeros((16,), F32)
        for h in range(nh):
            pltpu.sync_copy(scalT_h.at[pl.ds(h * N, N)], as_v)
            pltpu.sync_copy(scalT_h.at[pl.ds((nh + h) * N, N)], ad_v)

            def zbody(i, _):
                dn_v[pl.ds(i * 16, 16)] = zf
                return 0
            lax.fori_loop(0, N // 16, zbody, 0)

            @plsc.parallel_loop(0, NV, unroll=4)
            def vbody(j):
                o = j * 16
                srcv = src_c[pl.ds(o, 16)]
                dstv = dst_c[pl.ds(o, 16)]
                t0v = t0_c[pl.ds(o, 16)]
                t1v = t1_c[pl.ds(o, 16)]
                av = (plsc.load_gather(as_v, [srcv])
                      + plsc.load_gather(ad_v, [dstv])
                      + plsc.load_gather(ra_v, [t0v * 4 + h])
                      + plsc.load_gather(ra_v, [t1v * 4 + h]))
                av = jnp.maximum(av, 0.2 * av)
                sv = jnp.exp(av)
                s_c[pl.ds(o, 16)] = sv
                plsc.addupdate_scatter(dn_v, [srcv], sv)
            pltpu.sync_copy(s_c, s_out.at[pl.ds(h * ET + base, EW)])
            pltpu.sync_copy(dn_v, dp_out.at[pl.ds((wid * nh + h) * N, N)])

    return functools.partial(
        pl.kernel, body,
        out_type=(jax.ShapeDtypeStruct((nh * ET,), F32),
                  jax.ShapeDtypeStruct((NW * nh * N,), F32)),
        scratch_types=[
            pltpu.VMEM((N,), F32),
            pltpu.VMEM((N,), F32),
            pltpu.VMEM((N,), F32),
            pltpu.VMEM((TAB * 4,), F32),
            pltpu.VMEM((EW,), I32),
            pltpu.VMEM((EW,), I32),
            pltpu.VMEM((EW,), I32),
            pltpu.VMEM((EW,), I32),
            pltpu.VMEM((EW,), F32),
        ],
        **_SC_PARAMS)()


def _make_pass_b(nh):
    EW = ET // 2
    C = 1280
    NG = C // 128
    NCH = EW // C
    NVC = C * 8 // 16

    def body(sd_h, t01_h, s_h, xd2_h, tabG_h, zero_h,
             acc_out, acc_v, tab_v, sd_c, t01_c, s_c,
             idx_v, rows_v, msem, gsem):
        wid = lax.axis_index("s") * 2 + lax.axis_index("c")
        g = wid // 2
        half = wid - g * 2
        h = (g * nh) // 16
        pltpu.sync_copy(tabG_h.at[pl.ds(g * TAB * 8, TAB * 8)], tab_v)
        pltpu.sync_copy(zero_h, acc_v)
        lane = lax.iota(I32, 16)
        c01 = lane >> 3
        col = jnp.where(lane < 8, lane, 15 - lane)
        lane2p1 = lane * 2 + 1
        ebase = half * EW

        def _slot(c):
            return c - (c // 3) * 3

        def _par(c):
            return c - (c // 2) * 2

        def _meta_copies(c):
            cb = ebase + c * C
            slot, par = _slot(c), _par(c)
            return [
                pltpu.make_async_copy(sd_h.at[pl.ds(cb * 2, 2 * C)],
                                      sd_c.at[slot], msem.at[par]),
                pltpu.make_async_copy(t01_h.at[pl.ds(cb, C)],
                                      t01_c.at[slot], msem.at[par]),
                pltpu.make_async_copy(s_h.at[pl.ds(h * ET + cb, C)],
                                      s_c.at[slot], msem.at[par]),
            ]

        def meta_fire(c):
            for cp in _meta_copies(c):
                cp.start()

        def meta_wait(c):
            for cp in _meta_copies(c):
                cp.wait()

        def _gather_copies(c):
            par = _par(c)
            return [
                pltpu.make_async_copy(xd2_h.at[idx_v.at[par, k]],
                                      rows_v.at[par, pl.ds(k * 128, 128)],
                                      gsem.at[par])
                for k in range(NG)
            ]

        def gather_fire(c):
            slot, par = _slot(c), _par(c)
            ms = jnp.full((16,), slot, I32)

            @plsc.parallel_loop(0, C // 16, unroll=2)
            def ibody(j):
                dstv = plsc.load_gather(sd_c, [ms, j * 32 + lane2p1])
                row = j // 8
                off = (j - row * 8) * 16
                idx_v[par, row, pl.ds(off, 16)] = dstv * 16 + g
            for cp in _gather_copies(c):
                cp.start()

        def gather_wait(c):
            for cp in _gather_copies(c):
                cp.wait()

        meta_fire(0)
        meta_wait(0)
        gather_fire(0)
        meta_fire(1)

        def cbody(ci, _):
            @pl.when(ci + 1 < NCH)
            def _():
                meta_wait(ci + 1)
                gather_fire(ci + 1)

            @pl.when(ci + 2 < NCH)
            def _():
                meta_fire(ci + 2)

            gather_wait(ci)
            slot, par = _slot(ci), _par(ci)
            ms = jnp.full((16,), slot, I32)
            ps = jnp.full((16,), par, I32)

            @plsc.parallel_loop(0, NVC, unroll=4)
            def vbody(j):
                esp = c01 + j * 2
                srcp = plsc.load_gather(sd_c, [ms, esp * 2])
                sv = plsc.load_gather(s_c, [ms, esp])
                t01p = plsc.load_gather(t01_c, [ms, esp])
                tr0 = plsc.load_gather(tab_v, [(t01p >> 8) * 8 + col])
                tr1 = plsc.load_gather(tab_v, [(t01p & 255) * 8 + col])
                row = plsc.load_gather(rows_v, [ps, esp, col])
                val = (row + tr0 + tr1) * sv
                plsc.addupdate_scatter(acc_v, [srcp, col], val)
            return 0
        lax.fori_loop(0, NCH, cbody, 0)
        pltpu.sync_copy(acc_v, acc_out.at[g, half])

    return functools.partial(
        pl.kernel, body,
        out_type=jax.ShapeDtypeStruct((16, 2, N, 8), F32),
        scratch_types=[
            pltpu.VMEM((N, 8), F32),
            pltpu.VMEM((TAB * 8,), F32),
            pltpu.VMEM((3, 2 * C), I32),
            pltpu.VMEM((3, C), I32),
            pltpu.VMEM((3, C), F32),
            pltpu.VMEM((2, NG, 128), I32),
            pltpu.VMEM((2, C, 8), F32),
            pltpu.SemaphoreType.DMA((2,)),
            pltpu.SemaphoreType.DMA((2,)),
        ],
        **_SC_PARAMS)()


def _make_pass_c(nh):
    R = 400
    NWC = N // R

    def body(accp_h, xs_h, csum_h, recip_h, out_h, ab0, ab1, obuf, xsb,
             csb, rcb):
        wid = lax.axis_index("s") * 2 + lax.axis_index("c")

        @pl.when(wid < NWC)
        def _():
            base = wid * R
            lane = lax.iota(I32, 16)
            c01 = lane >> 3
            col = jnp.where(lane < 8, lane, 15 - lane)
            for h in range(nh):
                pltpu.sync_copy(csum_h.at[pl.ds(h * N + base, R)],
                                csb.at[pl.ds(h * R, R)])
                pltpu.sync_copy(recip_h.at[pl.ds(h * N + base, R)],
                                rcb.at[pl.ds(h * R, R)])
            for g in range(16):
                pltpu.sync_copy(
                    accp_h.at[pl.ds(((g * 2) * N + base) * 8, R * 8)], ab0)
                pltpu.sync_copy(
                    accp_h.at[pl.ds(((g * 2 + 1) * N + base) * 8, R * 8)], ab1)

                @plsc.parallel_loop(0, R * 8 // 16, unroll=4)
                def rbody(j):
                    esp = c01 + j * 2
                    v = (plsc.load_gather(ab0, [esp * 8 + col])
                         + plsc.load_gather(ab1, [esp * 8 + col]))
                    plsc.store_scatter(obuf, [esp, g * 8 + col], v)
            pltpu.sync_copy(xs_h.at[pl.ds(base * 128, R * 128)], xsb)

            @plsc.parallel_loop(0, R, unroll=2)
            def ebody(r):
                for k in range(8):
                    h = k // 2 if nh == 4 else 0
                    cs = csb[pl.ds(h * R + r, 16)][0]
                    rc = rcb[pl.ds(h * R + r, 16)][0]
                    v = obuf[r, pl.ds(k * 16, 16)]
                    xv = xsb[pl.ds(r * 128 + k * 16, 16)]
                    v = v * rc + xv * cs
                    v = jnp.where(v > 0.0, v, jnp.exp(v) - 1.0)
                    obuf[r, pl.ds(k * 16, 16)] = v
            pltpu.sync_copy(obuf, out_h.at[pl.ds(base, R)])

    return functools.partial(
        pl.kernel, body,
        out_type=jax.ShapeDtypeStruct((N, 128), F32),
        scratch_types=[
            pltpu.VMEM((R * 8,), F32),
            pltpu.VMEM((R * 8,), F32),
            pltpu.VMEM((R, 128), F32),
            pltpu.VMEM((R * 128,), F32),
            pltpu.VMEM((4 * R + 16,), F32),
            pltpu.VMEM((4 * R + 16,), F32),
        ],
        **_SC_PARAMS)()


_PASS_A = {nh: _make_pass_a(nh) for nh in (4, 1)}
_PASS_B = {nh: _make_pass_b(nh) for nh in (4, 1)}
_PASS_C = {nh: _make_pass_c(nh) for nh in (4, 1)}


def _layer(src, dst, t0, t1, sd, t01, x, Md, Ms, Ma, tab, ra, zero8, nh):
    xd, xs, scalT = _proj(x, Md, Ms, Ma)
    s, dp = _PASS_A[nh](src, dst, t0, t1, scalT.reshape(8 * N),
                        ra.reshape(TAB * 4))
    recip, csum = _merge(dp.reshape(NW, nh, N), nh)
    xd2 = xd.reshape(N * 16, 8)
    tabG = tab.reshape(TAB, 16, 8).transpose(1, 0, 2).reshape(16 * TAB * 8)
    accp = _PASS_B[nh](sd, t01, s, xd2, tabG, zero8)
    return _PASS_C[nh](accp.reshape(16 * 2 * N * 8), xs.reshape(N * 128),
                       csum.reshape(nh * N), recip.reshape(nh * N))


def kernel(edge_index, x, r, edge_type, edge_type_nhop,
           W0, a0, W1, a1, W2, a2, W3, a3, W_out, a_out, W_rel):
    edge_index = edge_index.astype(I32)
    src = edge_index[0]
    dst = edge_index[1]
    t0 = jnp.concatenate([edge_type.astype(I32),
                          edge_type_nhop[:, 0].astype(I32)])
    t1 = jnp.concatenate([jnp.full((E1,), NREL, I32),
                          edge_type_nhop[:, 1].astype(I32)])
    sd = jnp.stack([src, dst], axis=1).reshape(2 * ET)
    t01 = t0 * 256 + t1

    (Md1, Ms1, Ma1, tab1, ra1, Ma2, tab2, ra2, r2) = _prep(
        W0, a0, W1, a1, W2, a2, W3, a3, W_out, a_out, W_rel, r)

    zero8 = jnp.zeros((N, 8), F32)
    h = _layer(src, dst, t0, t1, sd, t01, x, Md1, Ms1, Ma1, tab1, ra1,
               zero8, 4)
    Ms2 = W_out[0:128]
    Md2 = W_out[128:256]
    out = _layer(src, dst, t0, t1, sd, t01, h, Md2, Ms2, Ma2, tab2, ra2,
                 zero8, 1)
    return (out, r2)
